# Initial kernel scaffold; baseline (speedup 1.0000x reference)
import functools
import math

import jax
import jax.numpy as jnp
from jax.experimental import pallas as pl

INTERPRET = False

CUTOFF = 5.0
P_ENV = 5
R = 6
S = 7
BIL = 8
H = 128

MASK3 = jnp.array([1.0, 1.0, 1.0, 0.0, 0.0, 0.0, 0.0, 0.0], jnp.float32)
ONEHOT3 = jnp.array([0.0, 0.0, 0.0, 1.0, 0.0, 0.0, 0.0, 0.0], jnp.float32)
FREQ8 = jnp.array([1.0, 2.0, 3.0, 4.0, 5.0, 6.0, 0.0, 0.0], jnp.float32) * math.pi
NMASK8 = jnp.array([1.0, 1.0, 1.0, 1.0, 1.0, 1.0, 0.0, 0.0], jnp.float32)


def _silu(x):
    return x * jax.nn.sigmoid(x)


def _envelope(x):
    p = P_ENV
    a = -(p + 1) * (p + 2) / 2.0
    b = float(p * (p + 2))
    c = -p * (p + 1) / 2.0
    xs = jnp.clip(x, 1e-6, None)
    env = 1.0 / xs + a * xs ** (p - 1) + b * xs ** p + c * xs ** (p + 1)
    return jnp.where(x < 1.0, env, 0.0)


def _rbf8(d):
    x = d / CUTOFF
    return _envelope(x) * jnp.sin(FREQ8[None, :] * x) * NMASK8[None, :]


def _prep_body(emb_table_ref, emb_W_ref, emb_Wd_ref, emb_bd_ref, emb_b_ref,
               out_Wd0_ref, t0_ref, t1_ref, wrd_ref, brd_ref, owd0_ref):
    tab = emb_table_ref[...]
    tab = jnp.concatenate([tab, jnp.zeros((33, H), jnp.float32)], axis=0)
    W = emb_W_ref[...]
    t0_ref[...] = tab @ W[0:H, :]
    t1_ref[...] = tab @ W[H:2 * H, :]
    wd = emb_Wd_ref[...]
    wd8 = jnp.concatenate([wd, jnp.zeros((8 - R, H), jnp.float32)], axis=0)
    wrd_ref[...] = wd8 @ W[2 * H:, :]
    brd_ref[...] = emb_bd_ref[...] @ W[2 * H:, :] + emb_b_ref[...]
    ow = out_Wd0_ref[...]
    owd0_ref[...] = jnp.concatenate([ow, jnp.zeros((8 - R, H), jnp.float32)],
                                    axis=0)


def _prep(emb_table, emb_W, emb_Wd, emb_bd, emb_b, out_Wd0):
    shapes = (
        jax.ShapeDtypeStruct((H, H), jnp.float32),
        jax.ShapeDtypeStruct((H, H), jnp.float32),
        jax.ShapeDtypeStruct((8, H), jnp.float32),
        jax.ShapeDtypeStruct((1, H), jnp.float32),
        jax.ShapeDtypeStruct((8, H), jnp.float32),
    )
    return pl.pallas_call(_prep_body, out_shape=shapes, interpret=INTERPRET)(
        emb_table, emb_W, emb_Wd.reshape(R, H), emb_bd.reshape(1, H),
        emb_b.reshape(1, H), out_Wd0)


def _edge_body(ps_ref, pd_ref, t0_ref, t1_ref, wrd_ref, brd_ref, owd0_ref,
               wsm_ref, bsm_ref, msg_ref, geom_ref, t_ref, q_ref):
    ps = ps_ref[...]
    pd = pd_ref[...]
    diff = ps - pd
    vec = diff * MASK3[None, :]
    d2 = jnp.sum(vec * vec, axis=1, keepdims=True) + 1e-12
    d = jnp.sqrt(d2)
    rbf = _rbf8(d)
    iota = jax.lax.broadcasted_iota(jnp.float32, (1, H), 1)
    an_s = jnp.sum(ps * ONEHOT3[None, :], axis=1, keepdims=True)
    an_d = jnp.sum(pd * ONEHOT3[None, :], axis=1, keepdims=True)
    oh_s = jnp.where(an_s == iota, 1.0, 0.0)
    oh_d = jnp.where(an_d == iota, 1.0, 0.0)
    pre = (oh_s @ t0_ref[...] + oh_d @ t1_ref[...] + rbf @ wrd_ref[...]
           + brd_ref[...])
    msg = _silu(pre)
    msg_ref[...] = msg
    geom_ref[...] = jnp.concatenate([vec + d * ONEHOT3[None, :], rbf], axis=1)
    t_ref[...] = (rbf @ owd0_ref[...]) * msg
    q_ref[...] = msg @ wsm_ref[...] + bsm_ref[...]


def _edge_stage(ps, pd, t0, t1, wrd, brd, owd0, wsm0, bsm0, be):
    E = ps.shape[0]
    grid = (E // be,)
    bspec_r = pl.BlockSpec((be, 8), lambda i: (i, 0))
    wspec = lambda s: pl.BlockSpec(s, lambda i: (0,) * len(s))
    out_shapes = (
        jax.ShapeDtypeStruct((E, H), jnp.float32),
        jax.ShapeDtypeStruct((E, 16), jnp.float32),
        jax.ShapeDtypeStruct((E, H), jnp.float32),
        jax.ShapeDtypeStruct((E, H), jnp.float32),
    )
    out_specs = (
        pl.BlockSpec((be, H), lambda i: (i, 0)),
        pl.BlockSpec((be, 16), lambda i: (i, 0)),
        pl.BlockSpec((be, H), lambda i: (i, 0)),
        pl.BlockSpec((be, H), lambda i: (i, 0)),
    )
    return pl.pallas_call(
        _edge_body,
        grid=grid,
        in_specs=[bspec_r, bspec_r, wspec((H, H)), wspec((H, H)),
                  wspec((8, H)), wspec((1, H)), wspec((8, H)),
                  wspec((H, H)), wspec((1, H))],
        out_specs=out_specs,
        out_shape=out_shapes,
        interpret=INTERPRET,
    )(ps, pd, t0, t1, wrd, brd, owd0, wsm0, bsm0)


def _sph_j(l, x):
    xs = jnp.clip(x, 1e-4, None)
    sx = jnp.sin(xs)
    cx = jnp.cos(xs)
    j0 = sx / xs
    if l == 0:
        return j0
    j1 = sx / (xs * xs) - cx / xs
    if l == 1:
        return j1
    jm1, jc = j0, j1
    for ll in range(1, l):
        jp = (2 * ll + 1) / xs * jc - jm1
        jm1, jc = jc, jp
    return jc


def _angle_body(gs_ref, gt_ref, wa_ref, amb_ref):
    gs = gs_ref[...]
    gt = gt_ref[...]
    vs = gs[:, 0:8] * MASK3[None, :]
    vt = gt[:, 0:8] * MASK3[None, :]
    ds = jnp.sum(gs[:, 0:8] * ONEHOT3[None, :], axis=1, keepdims=True)
    dt = jnp.sum(gt[:, 0:8] * ONEHOT3[None, :], axis=1, keepdims=True)
    cos = jnp.sum(vs * vt, axis=1, keepdims=True) / (ds * dt)
    cos = jnp.clip(cos, -1.0 + 1e-6, 1.0 - 1e-6)
    x = ds / CUTOFF
    env = _envelope(x)
    Ps = [jnp.ones_like(cos), cos]
    for l in range(1, S - 1):
        Ps.append(((2 * l + 1) * cos * Ps[l] - l * Ps[l - 1]) / (l + 1))
    n8 = jnp.array([1.0, 2.0, 3.0, 4.0, 5.0, 6.0, 1.0, 1.0], jnp.float32)
    feats = []
    for l in range(S):
        z = math.pi * (n8 + 0.5 * l)
        arg = z[None, :] * x
        feats.append(env * _sph_j(l, arg) * Ps[l] * NMASK8[None, :])
    sbf = jnp.concatenate(feats, axis=1)
    amb_ref[...] = sbf @ wa_ref[...]


def _angle_stage(gs, gt, wa_pad, ba):
    A = gs.shape[0]
    grid = (A // ba,)
    return pl.pallas_call(
        _angle_body,
        grid=grid,
        in_specs=[pl.BlockSpec((ba, 16), lambda i: (i, 0)),
                  pl.BlockSpec((ba, 16), lambda i: (i, 0)),
                  pl.BlockSpec((56, 32), lambda i: (0, 0))],
        out_specs=pl.BlockSpec((ba, 32), lambda i: (i, 0)),
        out_shape=jax.ShapeDtypeStruct((A, 32), jnp.float32),
        interpret=INTERPRET,
    )(gs, gt, wa_pad)


def _bilin_body(qse_ref, gt_ref, amb_ref, wd_ref, wt_ref, sm_ref, *, boff):
    rbf_te = gt_ref[...][:, 8:16]
    dmb = rbf_te @ wd_ref[...]
    smp = qse_ref[...] * dmb
    amb = amb_ref[...]
    wt = wt_ref[...]
    acc = None
    for j in range(BIL):
        term = amb[:, boff + j:boff + j + 1] * (smp @ wt[j])
        acc = term if acc is None else acc + term
    sm_ref[...] = acc


def _bilin_stage(qse, gt, amb, wd_b, wt_b, boff, ba):
    A = qse.shape[0]
    grid = (A // ba,)
    return pl.pallas_call(
        functools.partial(_bilin_body, boff=boff),
        grid=grid,
        in_specs=[pl.BlockSpec((ba, H), lambda i: (i, 0)),
                  pl.BlockSpec((ba, 16), lambda i: (i, 0)),
                  pl.BlockSpec((ba, 32), lambda i: (i, 0)),
                  pl.BlockSpec((8, H), lambda i: (0, 0)),
                  pl.BlockSpec((BIL, H, H), lambda i: (0, 0, 0))],
        out_specs=pl.BlockSpec((ba, H), lambda i: (i, 0)),
        out_shape=jax.ShapeDtypeStruct((A, H), jnp.float32),
        interpret=INTERPRET,
    )(qse, gt, amb, wd_b, wt_b)


def _emlp_body(agg_ref, msg_ref, geom_ref, wtm_ref, btm_ref,
               r1w1_ref, r1b1_ref, r1w2_ref, r1b2_ref,
               wsk_ref, bsk_ref,
               r2w1_ref, r2b1_ref, r2w2_ref, r2b2_ref,
               owd_ref, wsm_ref, bsm_ref,
               out_msg_ref, out_t_ref, out_q_ref, *, has_q):
    msg = msg_ref[...]
    x = agg_ref[...] + msg @ wtm_ref[...] + btm_ref[...]
    x = x + _silu(_silu(x @ r1w1_ref[...] + r1b1_ref[...]) @ r1w2_ref[...]
                  + r1b2_ref[...])
    x = _silu(x @ wsk_ref[...] + bsk_ref[...]) + msg
    for l in range(2):
        x = x + _silu(_silu(x @ r2w1_ref[l] + r2b1_ref[l]) @ r2w2_ref[l]
                      + r2b2_ref[l])
    out_msg_ref[...] = x
    rbf = geom_ref[...][:, 8:16]
    out_t_ref[...] = (rbf @ owd_ref[...]) * x
    if has_q:
        out_q_ref[...] = x @ wsm_ref[...] + bsm_ref[...]
    else:
        out_q_ref[...] = x


def _emlp_stage(agg, msg, geom, wtm, btm, r1w1, r1b1, r1w2, r1b2,
                wsk, bsk, r2w1, r2b1, r2w2, r2b2, owd, wsm, bsm, be):
    E = agg.shape[0]
    grid = (E // be,)
    row = lambda w: pl.BlockSpec((be, w), lambda i: (i, 0))
    wspec = lambda s: pl.BlockSpec(s, lambda i: (0,) * len(s))
    has_q = wsm is not None
    out_shapes = (
        jax.ShapeDtypeStruct((E, H), jnp.float32),
        jax.ShapeDtypeStruct((E, H), jnp.float32),
        jax.ShapeDtypeStruct((E, H), jnp.float32),
    )
    return pl.pallas_call(
        functools.partial(_emlp_body, has_q=has_q),
        grid=grid,
        in_specs=[row(H), row(H), row(16),
                  wspec((H, H)), wspec((1, H)),
                  wspec((H, H)), wspec((1, H)), wspec((H, H)), wspec((1, H)),
                  wspec((H, H)), wspec((1, H)),
                  wspec((2, H, H)), wspec((2, 1, H)), wspec((2, H, H)),
                  wspec((2, 1, H)),
                  wspec((8, H)), wspec((H, H)), wspec((1, H))],
        out_specs=(row(H), row(H), row(H)),
        out_shape=out_shapes,
        interpret=INTERPRET,
    )(agg, msg, geom, wtm, btm, r1w1, r1b1, r1w2, r1b2, wsk, bsk,
      r2w1, r2b1, r2w2, r2b2, owd,
      wsm if has_q else wtm, bsm if has_q else btm)


def _node_body(x_ref, wl_ref, bl_ref, wout_ref, out_ref, *, nb1):
    acc = None
    for b in range(nb1):
        x = x_ref[b]
        for l in range(3):
            x = _silu(x @ wl_ref[b, l] + bl_ref[b, l])
        o = x @ wout_ref[b]
        acc = o if acc is None else acc + o
    out_ref[...] = acc


def _node_stage(xs, out_Wl, out_bl, wout_pad, bn):
    nb1, N, _ = xs.shape
    grid = (N // bn,)
    return pl.pallas_call(
        functools.partial(_node_body, nb1=nb1),
        grid=grid,
        in_specs=[pl.BlockSpec((nb1, bn, H), lambda i: (0, i, 0)),
                  pl.BlockSpec(out_Wl.shape, lambda i: (0, 0, 0, 0)),
                  pl.BlockSpec((nb1, 3, 1, H), lambda i: (0, 0, 0, 0)),
                  pl.BlockSpec((nb1, H, 8), lambda i: (0, 0, 0))],
        out_specs=pl.BlockSpec((bn, 8), lambda i: (i, 0)),
        out_shape=jax.ShapeDtypeStruct((N, 8), jnp.float32),
        interpret=INTERPRET,
    )(xs, out_Wl, out_bl.reshape(nb1, 3, 1, H), wout_pad)


def kernel(atomic_number, position, edge_index, angle_index, emb_table,
           emb_Wd, emb_bd, emb_W, emb_b,
           int_Wd, int_Wa, int_Wsm, int_bsm, int_Wtm, int_btm, int_Wbil,
           int_res1_W1, int_res1_b1, int_res1_W2, int_res1_b2,
           int_Wskip, int_bskip,
           int_res2_W1, int_res2_b1, int_res2_W2, int_res2_b2,
           out_Wd, out_Wl, out_bl, out_Wout):
    N = position.shape[0]
    E = edge_index.shape[1]
    A = angle_index.shape[1]
    NB = int_Wd.shape[0]
    be = 1000 if E % 1000 == 0 else E
    ba = 1000 if A % 1000 == 0 else A
    bn = 1000 if N % 1000 == 0 else N

    src, dst = edge_index[0], edge_index[1]
    se, te = angle_index[0], angle_index[1]

    ptab = jnp.concatenate(
        [position, atomic_number[:, None].astype(jnp.float32),
         jnp.zeros((N, 4), jnp.float32)], axis=1)

    t0, t1, wrd, brd, owd0 = _prep(emb_table, emb_W, emb_Wd, emb_bd, emb_b,
                                   out_Wd[0])

    ps = jnp.take(ptab, src, axis=0)
    pd = jnp.take(ptab, dst, axis=0)

    wsm0 = int_Wsm[0]
    bsm0 = int_bsm[0].reshape(1, H)
    msg, geom, t_cur, q = _edge_stage(ps, pd, t0, t1, wrd, brd, owd0,
                                      wsm0, bsm0, be)

    gs = jnp.take(geom, se, axis=0)
    gt = jnp.take(geom, te, axis=0)

    wa = int_Wa.reshape(NB, S, R, BIL)
    wa = jnp.pad(wa, ((0, 0), (0, 0), (0, 2), (0, 0)))
    wa = wa.reshape(NB, 56, BIL).transpose(1, 0, 2).reshape(56, NB * BIL)
    wa_pad = jnp.pad(wa, ((0, 0), (0, 32 - NB * BIL)))
    amb = _angle_stage(gs, gt, wa_pad, ba)

    wt = int_Wbil.transpose(0, 2, 3, 1)
    wd8 = jnp.pad(int_Wd, ((0, 0), (0, 8 - R), (0, 0)))
    owd8 = jnp.pad(out_Wd, ((0, 0), (0, 8 - R), (0, 0)))
    wout_pad = jnp.pad(out_Wout, ((0, 0), (0, 0), (0, 8 - out_Wout.shape[2])))

    t_list = [t_cur]
    for b in range(NB):
        qse = jnp.take(q, se, axis=0)
        sm = _bilin_stage(qse, gt, amb, wd8[b], wt[b], BIL * b, ba)
        agg = jax.ops.segment_sum(sm, te, num_segments=E)
        has_q = b + 1 < NB
        msg, t_cur, q = _emlp_stage(
            agg, msg, geom, int_Wtm[b], int_btm[b].reshape(1, H),
            int_res1_W1[b, 0], int_res1_b1[b, 0].reshape(1, H),
            int_res1_W2[b, 0], int_res1_b2[b, 0].reshape(1, H),
            int_Wskip[b], int_bskip[b].reshape(1, H),
            int_res2_W1[b], int_res2_b1[b].reshape(2, 1, H),
            int_res2_W2[b], int_res2_b2[b].reshape(2, 1, H),
            owd8[b + 1],
            int_Wsm[b + 1] if has_q else None,
            int_bsm[b + 1].reshape(1, H) if has_q else None, be)
        t_list.append(t_cur)

    xs = jnp.stack([jax.ops.segment_sum(t, dst, num_segments=N)
                    for t in t_list], axis=0)
    out = _node_stage(xs, out_Wl, out_bl, wout_pad, bn)
    return out[:, 0:1]


# speedup vs baseline: 1.2754x; 1.2754x over previous
import functools
import math

import jax
import jax.numpy as jnp
from jax.experimental import pallas as pl

INTERPRET = False

CUTOFF = 5.0
P_ENV = 5
R = 6
S = 7
BIL = 8
H = 128

def _lane_consts():
    lane = jax.lax.broadcasted_iota(jnp.int32, (1, 8), 1).astype(jnp.float32)
    mask3 = jnp.where(lane < 3.0, 1.0, 0.0)
    onehot3 = jnp.where(lane == 3.0, 1.0, 0.0)
    nmask8 = jnp.where(lane < 6.0, 1.0, 0.0)
    return lane, mask3, onehot3, nmask8


def _silu(x):
    return x * jax.nn.sigmoid(x)


def _envelope(x):
    p = P_ENV
    a = -(p + 1) * (p + 2) / 2.0
    b = float(p * (p + 2))
    c = -p * (p + 1) / 2.0
    xs = jnp.clip(x, 1e-6, None)
    env = 1.0 / xs + a * xs ** (p - 1) + b * xs ** p + c * xs ** (p + 1)
    return jnp.where(x < 1.0, env, 0.0)


def _rbf8(d, lane, nmask8):
    x = d / CUTOFF
    freq8 = (lane + 1.0) * math.pi * nmask8
    return _envelope(x) * jnp.sin(freq8 * x) * nmask8


def _prep_body(emb_table_ref, emb_W_ref, emb_Wd_ref, emb_bd_ref, emb_b_ref,
               out_Wd0_ref, t0_ref, t1_ref, wrd_ref, brd_ref, owd0_ref):
    tab = emb_table_ref[...]
    tab = jnp.concatenate([tab, jnp.zeros((33, H), jnp.float32)], axis=0)
    W = emb_W_ref[...]
    t0_ref[...] = tab @ W[0:H, :]
    t1_ref[...] = tab @ W[H:2 * H, :]
    wd = emb_Wd_ref[...]
    wd8 = jnp.concatenate([wd, jnp.zeros((8 - R, H), jnp.float32)], axis=0)
    wrd_ref[...] = wd8 @ W[2 * H:, :]
    brd_ref[...] = emb_bd_ref[...] @ W[2 * H:, :] + emb_b_ref[...]
    ow = out_Wd0_ref[...]
    owd0_ref[...] = jnp.concatenate([ow, jnp.zeros((8 - R, H), jnp.float32)],
                                    axis=0)


def _prep(emb_table, emb_W, emb_Wd, emb_bd, emb_b, out_Wd0):
    shapes = (
        jax.ShapeDtypeStruct((H, H), jnp.float32),
        jax.ShapeDtypeStruct((H, H), jnp.float32),
        jax.ShapeDtypeStruct((8, H), jnp.float32),
        jax.ShapeDtypeStruct((1, H), jnp.float32),
        jax.ShapeDtypeStruct((8, H), jnp.float32),
    )
    return pl.pallas_call(_prep_body, out_shape=shapes, interpret=INTERPRET)(
        emb_table, emb_W, emb_Wd.reshape(R, H), emb_bd.reshape(1, H),
        emb_b.reshape(1, H), out_Wd0)


def _edge_body(ps_ref, pd_ref, t0_ref, t1_ref, wrd_ref, brd_ref, owd0_ref,
               wsm_ref, bsm_ref, msg_ref, geom_ref, t_ref, q_ref):
    lane, mask3, onehot3, nmask8 = _lane_consts()
    ps = ps_ref[...]
    pd = pd_ref[...]
    diff = ps - pd
    vec = diff * mask3
    d2 = jnp.sum(vec * vec, axis=1, keepdims=True) + 1e-12
    d = jnp.sqrt(d2)
    rbf = _rbf8(d, lane, nmask8)
    iota = jax.lax.broadcasted_iota(jnp.int32, (1, H), 1).astype(jnp.float32)
    an_s = jnp.sum(ps * onehot3, axis=1, keepdims=True)
    an_d = jnp.sum(pd * onehot3, axis=1, keepdims=True)
    oh_s = jnp.where(an_s == iota, 1.0, 0.0)
    oh_d = jnp.where(an_d == iota, 1.0, 0.0)
    pre = (oh_s @ t0_ref[...] + oh_d @ t1_ref[...] + rbf @ wrd_ref[...]
           + brd_ref[...])
    msg = _silu(pre)
    msg_ref[...] = msg
    geom_ref[...] = jnp.concatenate([vec + d * onehot3, rbf], axis=1)
    t_ref[...] = (rbf @ owd0_ref[...]) * msg
    q_ref[...] = msg @ wsm_ref[...] + bsm_ref[...]


def _edge_stage(ps, pd, t0, t1, wrd, brd, owd0, wsm0, bsm0, be):
    E = ps.shape[0]
    grid = (E // be,)
    bspec_r = pl.BlockSpec((be, 8), lambda i: (i, 0))
    wspec = lambda s: pl.BlockSpec(s, lambda i: (0,) * len(s))
    out_shapes = (
        jax.ShapeDtypeStruct((E, H), jnp.float32),
        jax.ShapeDtypeStruct((E, 16), jnp.float32),
        jax.ShapeDtypeStruct((E, H), jnp.float32),
        jax.ShapeDtypeStruct((E, H), jnp.float32),
    )
    out_specs = (
        pl.BlockSpec((be, H), lambda i: (i, 0)),
        pl.BlockSpec((be, 16), lambda i: (i, 0)),
        pl.BlockSpec((be, H), lambda i: (i, 0)),
        pl.BlockSpec((be, H), lambda i: (i, 0)),
    )
    return pl.pallas_call(
        _edge_body,
        grid=grid,
        in_specs=[bspec_r, bspec_r, wspec((H, H)), wspec((H, H)),
                  wspec((8, H)), wspec((1, H)), wspec((8, H)),
                  wspec((H, H)), wspec((1, H))],
        out_specs=out_specs,
        out_shape=out_shapes,
        interpret=INTERPRET,
    )(ps, pd, t0, t1, wrd, brd, owd0, wsm0, bsm0)


def _sph_j(l, x):
    xs = jnp.clip(x, 1e-4, None)
    sx = jnp.sin(xs)
    cx = jnp.cos(xs)
    j0 = sx / xs
    if l == 0:
        return j0
    j1 = sx / (xs * xs) - cx / xs
    if l == 1:
        return j1
    jm1, jc = j0, j1
    for ll in range(1, l):
        jp = (2 * ll + 1) / xs * jc - jm1
        jm1, jc = jc, jp
    return jc


def _angle_body(gs_ref, gt_ref, wa_ref, amb_ref):
    lane, mask3, onehot3, nmask8 = _lane_consts()
    gs = gs_ref[...]
    gt = gt_ref[...]
    vs = gs[:, 0:8] * mask3
    vt = gt[:, 0:8] * mask3
    ds = jnp.sum(gs[:, 0:8] * onehot3, axis=1, keepdims=True)
    dt = jnp.sum(gt[:, 0:8] * onehot3, axis=1, keepdims=True)
    cos = jnp.sum(vs * vt, axis=1, keepdims=True) / (ds * dt)
    cos = jnp.clip(cos, -1.0 + 1e-6, 1.0 - 1e-6)
    x = ds / CUTOFF
    env = _envelope(x)
    Ps = [jnp.ones_like(cos), cos]
    for l in range(1, S - 1):
        Ps.append(((2 * l + 1) * cos * Ps[l] - l * Ps[l - 1]) / (l + 1))
    n8 = jnp.where(lane < 6.0, lane + 1.0, 1.0)
    feats = []
    for l in range(S):
        z = math.pi * (n8 + 0.5 * l)
        arg = z * x
        feats.append(env * _sph_j(l, arg) * Ps[l] * nmask8)
    sbf = jnp.concatenate(feats, axis=1)
    amb_ref[...] = sbf @ wa_ref[...]


def _angle_stage(gs, gt, wa_pad, ba):
    A = gs.shape[0]
    grid = (A // ba,)
    return pl.pallas_call(
        _angle_body,
        grid=grid,
        in_specs=[pl.BlockSpec((ba, 16), lambda i: (i, 0)),
                  pl.BlockSpec((ba, 16), lambda i: (i, 0)),
                  pl.BlockSpec((56, 32), lambda i: (0, 0))],
        out_specs=pl.BlockSpec((ba, 32), lambda i: (i, 0)),
        out_shape=jax.ShapeDtypeStruct((A, 32), jnp.float32),
        interpret=INTERPRET,
    )(gs, gt, wa_pad)


def _bilin_body(qse_ref, gt_ref, amb_ref, wd_ref, wt_ref, sm_ref, *, boff):
    rbf_te = gt_ref[...][:, 8:16]
    dmb = rbf_te @ wd_ref[...]
    smp = qse_ref[...] * dmb
    amb = amb_ref[...]
    wt = wt_ref[...]
    acc = None
    for j in range(BIL):
        term = amb[:, boff + j:boff + j + 1] * (smp @ wt[j])
        acc = term if acc is None else acc + term
    sm_ref[...] = acc


def _bilin_stage(qse, gt, amb, wd_b, wt_b, boff, ba):
    A = qse.shape[0]
    grid = (A // ba,)
    return pl.pallas_call(
        functools.partial(_bilin_body, boff=boff),
        grid=grid,
        in_specs=[pl.BlockSpec((ba, H), lambda i: (i, 0)),
                  pl.BlockSpec((ba, 16), lambda i: (i, 0)),
                  pl.BlockSpec((ba, 32), lambda i: (i, 0)),
                  pl.BlockSpec((8, H), lambda i: (0, 0)),
                  pl.BlockSpec((BIL, H, H), lambda i: (0, 0, 0))],
        out_specs=pl.BlockSpec((ba, H), lambda i: (i, 0)),
        out_shape=jax.ShapeDtypeStruct((A, H), jnp.float32),
        interpret=INTERPRET,
    )(qse, gt, amb, wd_b, wt_b)


def _emlp_body(agg_ref, msg_ref, geom_ref, wtm_ref, btm_ref,
               r1w1_ref, r1b1_ref, r1w2_ref, r1b2_ref,
               wsk_ref, bsk_ref,
               r2w1_ref, r2b1_ref, r2w2_ref, r2b2_ref,
               owd_ref, wsm_ref, bsm_ref,
               out_msg_ref, out_t_ref, out_q_ref, *, has_q):
    msg = msg_ref[...]
    x = agg_ref[...] + msg @ wtm_ref[...] + btm_ref[...]
    x = x + _silu(_silu(x @ r1w1_ref[...] + r1b1_ref[...]) @ r1w2_ref[...]
                  + r1b2_ref[...])
    x = _silu(x @ wsk_ref[...] + bsk_ref[...]) + msg
    for l in range(2):
        x = x + _silu(_silu(x @ r2w1_ref[l] + r2b1_ref[l]) @ r2w2_ref[l]
                      + r2b2_ref[l])
    out_msg_ref[...] = x
    rbf = geom_ref[...][:, 8:16]
    out_t_ref[...] = (rbf @ owd_ref[...]) * x
    if has_q:
        out_q_ref[...] = x @ wsm_ref[...] + bsm_ref[...]
    else:
        out_q_ref[...] = x


def _emlp_stage(agg, msg, geom, wtm, btm, r1w1, r1b1, r1w2, r1b2,
                wsk, bsk, r2w1, r2b1, r2w2, r2b2, owd, wsm, bsm, be):
    E = agg.shape[0]
    grid = (E // be,)
    row = lambda w: pl.BlockSpec((be, w), lambda i: (i, 0))
    wspec = lambda s: pl.BlockSpec(s, lambda i: (0,) * len(s))
    has_q = wsm is not None
    out_shapes = (
        jax.ShapeDtypeStruct((E, H), jnp.float32),
        jax.ShapeDtypeStruct((E, H), jnp.float32),
        jax.ShapeDtypeStruct((E, H), jnp.float32),
    )
    return pl.pallas_call(
        functools.partial(_emlp_body, has_q=has_q),
        grid=grid,
        in_specs=[row(H), row(H), row(16),
                  wspec((H, H)), wspec((1, H)),
                  wspec((H, H)), wspec((1, H)), wspec((H, H)), wspec((1, H)),
                  wspec((H, H)), wspec((1, H)),
                  wspec((2, H, H)), wspec((2, 1, H)), wspec((2, H, H)),
                  wspec((2, 1, H)),
                  wspec((8, H)), wspec((H, H)), wspec((1, H))],
        out_specs=(row(H), row(H), row(H)),
        out_shape=out_shapes,
        interpret=INTERPRET,
    )(agg, msg, geom, wtm, btm, r1w1, r1b1, r1w2, r1b2, wsk, bsk,
      r2w1, r2b1, r2w2, r2b2, owd,
      wsm if has_q else wtm, bsm if has_q else btm)


def _node_body(x_ref, wl_ref, bl_ref, wout_ref, out_ref, *, nb1):
    acc = None
    for b in range(nb1):
        x = x_ref[b]
        for l in range(3):
            x = _silu(x @ wl_ref[b, l] + bl_ref[b, l])
        o = x @ wout_ref[b]
        acc = o if acc is None else acc + o
    out_ref[...] = acc


def _node_stage(xs, out_Wl, out_bl, wout_pad, bn):
    nb1, N, _ = xs.shape
    grid = (N // bn,)
    return pl.pallas_call(
        functools.partial(_node_body, nb1=nb1),
        grid=grid,
        in_specs=[pl.BlockSpec((nb1, bn, H), lambda i: (0, i, 0)),
                  pl.BlockSpec(out_Wl.shape, lambda i: (0, 0, 0, 0)),
                  pl.BlockSpec((nb1, 3, 1, H), lambda i: (0, 0, 0, 0)),
                  pl.BlockSpec((nb1, H, 8), lambda i: (0, 0, 0))],
        out_specs=pl.BlockSpec((bn, 8), lambda i: (i, 0)),
        out_shape=jax.ShapeDtypeStruct((N, 8), jnp.float32),
        interpret=INTERPRET,
    )(xs, out_Wl, out_bl.reshape(nb1, 3, 1, H), wout_pad)


def kernel(atomic_number, position, edge_index, angle_index, emb_table,
           emb_Wd, emb_bd, emb_W, emb_b,
           int_Wd, int_Wa, int_Wsm, int_bsm, int_Wtm, int_btm, int_Wbil,
           int_res1_W1, int_res1_b1, int_res1_W2, int_res1_b2,
           int_Wskip, int_bskip,
           int_res2_W1, int_res2_b1, int_res2_W2, int_res2_b2,
           out_Wd, out_Wl, out_bl, out_Wout):
    N = position.shape[0]
    E = edge_index.shape[1]
    A = angle_index.shape[1]
    NB = int_Wd.shape[0]
    be = 1000 if E % 1000 == 0 else E
    ba = 1000 if A % 1000 == 0 else A
    bn = 1000 if N % 1000 == 0 else N

    src, dst = edge_index[0], edge_index[1]
    se, te = angle_index[0], angle_index[1]

    ptab = jnp.concatenate(
        [position, atomic_number[:, None].astype(jnp.float32),
         jnp.zeros((N, 4), jnp.float32)], axis=1)

    t0, t1, wrd, brd, owd0 = _prep(emb_table, emb_W, emb_Wd, emb_bd, emb_b,
                                   out_Wd[0])

    ps = jnp.take(ptab, src, axis=0)
    pd = jnp.take(ptab, dst, axis=0)

    wsm0 = int_Wsm[0]
    bsm0 = int_bsm[0].reshape(1, H)
    msg, geom, t_cur, q = _edge_stage(ps, pd, t0, t1, wrd, brd, owd0,
                                      wsm0, bsm0, be)

    gs = jnp.take(geom, se, axis=0)
    gt = jnp.take(geom, te, axis=0)

    wa = int_Wa.reshape(NB, S, R, BIL)
    wa = jnp.pad(wa, ((0, 0), (0, 0), (0, 2), (0, 0)))
    wa = wa.reshape(NB, 56, BIL).transpose(1, 0, 2).reshape(56, NB * BIL)
    wa_pad = jnp.pad(wa, ((0, 0), (0, 32 - NB * BIL)))
    amb = _angle_stage(gs, gt, wa_pad, ba)

    wt = int_Wbil.transpose(0, 2, 3, 1)
    wd8 = jnp.pad(int_Wd, ((0, 0), (0, 8 - R), (0, 0)))
    owd8 = jnp.pad(out_Wd, ((0, 0), (0, 8 - R), (0, 0)))
    wout_pad = jnp.pad(out_Wout, ((0, 0), (0, 0), (0, 8 - out_Wout.shape[2])))

    t_list = [t_cur]
    for b in range(NB):
        qse = jnp.take(q, se, axis=0)
        sm = _bilin_stage(qse, gt, amb, wd8[b], wt[b], BIL * b, ba)
        agg = jax.ops.segment_sum(sm, te, num_segments=E)
        has_q = b + 1 < NB
        msg, t_cur, q = _emlp_stage(
            agg, msg, geom, int_Wtm[b], int_btm[b].reshape(1, H),
            int_res1_W1[b, 0], int_res1_b1[b, 0].reshape(1, H),
            int_res1_W2[b, 0], int_res1_b2[b, 0].reshape(1, H),
            int_Wskip[b], int_bskip[b].reshape(1, H),
            int_res2_W1[b], int_res2_b1[b].reshape(2, 1, H),
            int_res2_W2[b], int_res2_b2[b].reshape(2, 1, H),
            owd8[b + 1],
            int_Wsm[b + 1] if has_q else None,
            int_bsm[b + 1].reshape(1, H) if has_q else None, be)
        t_list.append(t_cur)

    xs = jnp.stack([jax.ops.segment_sum(t, dst, num_segments=N)
                    for t in t_list], axis=0)
    out = _node_stage(xs, out_Wl, out_bl, wout_pad, bn)
    return out[:, 0:1]


# speedup vs baseline: 1.8288x; 1.4339x over previous
import functools
import math

import jax
import jax.numpy as jnp
from jax import lax
from jax.experimental import pallas as pl
from jax.experimental.pallas import tpu as pltpu
from jax.experimental.pallas import tpu_sc as plsc

INTERPRET = False

_SC_NC = 2
_SC_NS = 16
_SC_NW = _SC_NC * _SC_NS


def _chunks_of(n, cb=128):
    out = [(i * cb, cb) for i in range(n // cb)]
    if n % cb:
        out.append(((n // cb) * cb, n % cb))
    return out


def _make_sc_gather(V, D, B, n_idx):
    del V
    assert B % (8 * _SC_NW) == 0
    bpw = B // _SC_NW
    mesh = plsc.VectorSubcoreMesh(core_axis_name="c", subcore_axis_name="s")
    chunks = _chunks_of(bpw)

    out_type = [jax.ShapeDtypeStruct((B, D), jnp.float32)
                for _ in range(n_idx)]
    scratch = [pltpu.VMEM((bpw,), jnp.int32) for _ in range(n_idx)]
    scratch += [pltpu.VMEM((128, D), jnp.float32), pltpu.VMEM((128, D), jnp.float32),
                pltpu.SemaphoreType.DMA, pltpu.SemaphoreType.DMA]

    @functools.partial(
        pl.kernel, out_type=out_type, mesh=mesh, scratch_types=scratch,
        compiler_params=pltpu.CompilerParams(use_tc_tiling_on_sc=False),
        name=f"sc_gather_{D}x{n_idx}")
    def gather_kernel(*refs):
        tabs = refs[:n_idx]
        idxs = refs[n_idx:2 * n_idx]
        outs = refs[2 * n_idx:3 * n_idx]
        sc = refs[3 * n_idx:]
        idxv = sc[:n_idx]
        rows = sc[n_idx:n_idx + 2]
        sems = sc[n_idx + 2:n_idx + 4]
        wid = lax.axis_index("s") * _SC_NC + lax.axis_index("c")
        base = wid * bpw
        for k in range(n_idx):
            pltpu.sync_copy(idxs[k].at[pl.ds(base, bpw)], idxv[k])
        work = [(k, off, sz) for k in range(n_idx) for (off, sz) in chunks]

        def start(w, slot):
            k, off, sz = work[w]
            return pltpu.async_copy(
                tabs[k].at[idxv[k].at[pl.ds(off, sz)]],
                rows[slot].at[pl.ds(0, sz)], sems[slot])

        pend = [None, None]
        pend[0] = start(0, 0)
        for w in range(len(work)):
            slot = w & 1
            if w + 1 < len(work):
                pend[1 - slot] = start(w + 1, 1 - slot)
            pend[slot].wait()
            k, off, sz = work[w]
            pltpu.sync_copy(rows[slot].at[pl.ds(0, sz)],
                            outs[k].at[pl.ds(base + off, sz)])

    return gather_kernel


def _make_sc_scatter_n(N, D, E):
    assert E % (8 * _SC_NW) == 0
    bpw = E // _SC_NW
    mesh = plsc.VectorSubcoreMesh(core_axis_name="c", subcore_axis_name="s")
    nfull, tail = bpw // 128, bpw % 128
    assert tail % 8 == 0 and tail > 0
    nt_rows = N // _SC_NS
    assert N % _SC_NS == 0

    @functools.partial(
        pl.kernel,
        out_type=jax.ShapeDtypeStruct((_SC_NC, N, D), jnp.float32),
        mesh=mesh,
        scratch_types=[
            pltpu.VMEM_SHARED((N, D), jnp.float32),
            pltpu.VMEM((128, D), jnp.float32),
            pltpu.VMEM((128, D), jnp.float32),
            pltpu.VMEM((128,), jnp.int32),
            pltpu.VMEM((128,), jnp.int32),
            pltpu.VMEM((tail, D), jnp.float32),
            pltpu.VMEM((tail,), jnp.int32),
            pltpu.SemaphoreType.DMA,
            pltpu.SemaphoreType.DMA,
            pltpu.SemaphoreType.DMA,
            pltpu.SemaphoreType.DMA,
        ],
        compiler_params=pltpu.CompilerParams(use_tc_tiling_on_sc=False),
        name="sc_scatter_n",
    )
    def scatter_kernel(t_hbm, dst_hbm, z_hbm, out_hbm, acc, rows0, rows1,
                       idx0, idx1, rows_t, idx_t, sem0, sem1, sem2, sem3):
        cid = lax.axis_index("c")
        sid = lax.axis_index("s")
        wid = sid * _SC_NC + cid
        base = wid * bpw
        pltpu.sync_copy(z_hbm, acc.at[pl.ds(sid * nt_rows, nt_rows)])
        plsc.subcore_barrier()
        rows = (rows0, rows1)
        idxs = (idx0, idx1)
        rsem = (sem0, sem1)
        isem = (sem2, sem3)

        def start(w, slot):
            off = w * 128
            c1 = pltpu.async_copy(t_hbm.at[pl.ds(base + off, 128)],
                                  rows[slot], rsem[slot])
            c2 = pltpu.async_copy(dst_hbm.at[pl.ds(base + off, 128)],
                                  idxs[slot], isem[slot])
            return c1, c2

        pend = [None, None]
        pend[0] = start(0, 0)
        for w in range(nfull):
            slot = w & 1
            if w + 1 < nfull:
                pend[1 - slot] = start(w + 1, 1 - slot)
            c1, c2 = pend[slot]
            c1.wait()
            c2.wait()
            pltpu.sync_copy(rows[slot], acc.at[idxs[slot]], add=True)
        if tail:
            toff = base + nfull * 128
            pltpu.sync_copy(t_hbm.at[pl.ds(toff, tail)], rows_t)
            pltpu.sync_copy(dst_hbm.at[pl.ds(toff, tail)], idx_t)
            pltpu.sync_copy(rows_t, acc.at[idx_t], add=True)
        plsc.subcore_barrier()
        pltpu.sync_copy(acc.at[pl.ds(sid * nt_rows, nt_rows)],
                        out_hbm.at[cid].at[pl.ds(sid * nt_rows, nt_rows)])

    return scatter_kernel

CUTOFF = 5.0
P_ENV = 5
R = 6
S = 7
BIL = 8
H = 128

def _lane_consts():
    lane = jax.lax.broadcasted_iota(jnp.int32, (1, 8), 1).astype(jnp.float32)
    mask3 = jnp.where(lane < 3.0, 1.0, 0.0)
    onehot3 = jnp.where(lane == 3.0, 1.0, 0.0)
    nmask8 = jnp.where(lane < 6.0, 1.0, 0.0)
    return lane, mask3, onehot3, nmask8


def _silu(x):
    return x * jax.nn.sigmoid(x)


def _envelope(x):
    p = P_ENV
    a = -(p + 1) * (p + 2) / 2.0
    b = float(p * (p + 2))
    c = -p * (p + 1) / 2.0
    xs = jnp.clip(x, 1e-6, None)
    env = 1.0 / xs + a * xs ** (p - 1) + b * xs ** p + c * xs ** (p + 1)
    return jnp.where(x < 1.0, env, 0.0)


def _rbf8(d, lane, nmask8):
    x = d / CUTOFF
    freq8 = (lane + 1.0) * math.pi * nmask8
    return _envelope(x) * jnp.sin(freq8 * x) * nmask8


def _prep_body(emb_table_ref, emb_W_ref, emb_Wd_ref, emb_bd_ref, emb_b_ref,
               out_Wd0_ref, t0_ref, t1_ref, wrd_ref, brd_ref, owd0_ref):
    tab = emb_table_ref[...]
    tab = jnp.concatenate([tab, jnp.zeros((33, H), jnp.float32)], axis=0)
    W = emb_W_ref[...]
    t0_ref[...] = tab @ W[0:H, :]
    t1_ref[...] = tab @ W[H:2 * H, :]
    wd = emb_Wd_ref[...]
    wd8 = jnp.concatenate([wd, jnp.zeros((8 - R, H), jnp.float32)], axis=0)
    wrd_ref[...] = wd8 @ W[2 * H:, :]
    brd_ref[...] = emb_bd_ref[...] @ W[2 * H:, :] + emb_b_ref[...]
    ow = out_Wd0_ref[...]
    owd0_ref[...] = jnp.concatenate([ow, jnp.zeros((8 - R, H), jnp.float32)],
                                    axis=0)


def _prep(emb_table, emb_W, emb_Wd, emb_bd, emb_b, out_Wd0):
    shapes = (
        jax.ShapeDtypeStruct((H, H), jnp.float32),
        jax.ShapeDtypeStruct((H, H), jnp.float32),
        jax.ShapeDtypeStruct((8, H), jnp.float32),
        jax.ShapeDtypeStruct((1, H), jnp.float32),
        jax.ShapeDtypeStruct((8, H), jnp.float32),
    )
    return pl.pallas_call(_prep_body, out_shape=shapes, interpret=INTERPRET)(
        emb_table, emb_W, emb_Wd.reshape(R, H), emb_bd.reshape(1, H),
        emb_b.reshape(1, H), out_Wd0)


def _edge_body(ps_ref, pd_ref, t0_ref, t1_ref, wrd_ref, brd_ref, owd0_ref,
               wsm_ref, bsm_ref, msg_ref, geom_ref, t_ref, q_ref):
    lane, mask3, onehot3, nmask8 = _lane_consts()
    ps = ps_ref[...]
    pd = pd_ref[...]
    diff = ps - pd
    vec = diff * mask3
    d2 = jnp.sum(vec * vec, axis=1, keepdims=True) + 1e-12
    d = jnp.sqrt(d2)
    rbf = _rbf8(d, lane, nmask8)
    iota = jax.lax.broadcasted_iota(jnp.int32, (1, H), 1).astype(jnp.float32)
    an_s = jnp.sum(ps * onehot3, axis=1, keepdims=True)
    an_d = jnp.sum(pd * onehot3, axis=1, keepdims=True)
    oh_s = jnp.where(an_s == iota, 1.0, 0.0)
    oh_d = jnp.where(an_d == iota, 1.0, 0.0)
    pre = (oh_s @ t0_ref[...] + oh_d @ t1_ref[...] + rbf @ wrd_ref[...]
           + brd_ref[...])
    msg = _silu(pre)
    msg_ref[...] = msg
    geom_ref[...] = jnp.concatenate([vec + d * onehot3, rbf], axis=1)
    t_ref[...] = (rbf @ owd0_ref[...]) * msg
    q_ref[...] = msg @ wsm_ref[...] + bsm_ref[...]


def _edge_stage(ps, pd, t0, t1, wrd, brd, owd0, wsm0, bsm0, be):
    E = ps.shape[0]
    grid = (E // be,)
    bspec_r = pl.BlockSpec((be, 8), lambda i: (i, 0))
    wspec = lambda s: pl.BlockSpec(s, lambda i: (0,) * len(s))
    out_shapes = (
        jax.ShapeDtypeStruct((E, H), jnp.float32),
        jax.ShapeDtypeStruct((E, 16), jnp.float32),
        jax.ShapeDtypeStruct((E, H), jnp.float32),
        jax.ShapeDtypeStruct((E, H), jnp.float32),
    )
    out_specs = (
        pl.BlockSpec((be, H), lambda i: (i, 0)),
        pl.BlockSpec((be, 16), lambda i: (i, 0)),
        pl.BlockSpec((be, H), lambda i: (i, 0)),
        pl.BlockSpec((be, H), lambda i: (i, 0)),
    )
    return pl.pallas_call(
        _edge_body,
        grid=grid,
        in_specs=[bspec_r, bspec_r, wspec((H, H)), wspec((H, H)),
                  wspec((8, H)), wspec((1, H)), wspec((8, H)),
                  wspec((H, H)), wspec((1, H))],
        out_specs=out_specs,
        out_shape=out_shapes,
        interpret=INTERPRET,
    )(ps, pd, t0, t1, wrd, brd, owd0, wsm0, bsm0)


def _sph_j(l, x):
    xs = jnp.clip(x, 1e-4, None)
    sx = jnp.sin(xs)
    cx = jnp.cos(xs)
    j0 = sx / xs
    if l == 0:
        return j0
    j1 = sx / (xs * xs) - cx / xs
    if l == 1:
        return j1
    jm1, jc = j0, j1
    for ll in range(1, l):
        jp = (2 * ll + 1) / xs * jc - jm1
        jm1, jc = jc, jp
    return jc


def _angle_body(gs_ref, gt_ref, wa_ref, amb_ref):
    lane, mask3, onehot3, nmask8 = _lane_consts()
    gs = gs_ref[...]
    gt = gt_ref[...]
    vs = gs[:, 0:8] * mask3
    vt = gt[:, 0:8] * mask3
    ds = jnp.sum(gs[:, 0:8] * onehot3, axis=1, keepdims=True)
    dt = jnp.sum(gt[:, 0:8] * onehot3, axis=1, keepdims=True)
    cos = jnp.sum(vs * vt, axis=1, keepdims=True) / (ds * dt)
    cos = jnp.clip(cos, -1.0 + 1e-6, 1.0 - 1e-6)
    x = ds / CUTOFF
    env = _envelope(x)
    Ps = [jnp.ones_like(cos), cos]
    for l in range(1, S - 1):
        Ps.append(((2 * l + 1) * cos * Ps[l] - l * Ps[l - 1]) / (l + 1))
    n8 = jnp.where(lane < 6.0, lane + 1.0, 1.0)
    feats = []
    for l in range(S):
        z = math.pi * (n8 + 0.5 * l)
        arg = z * x
        feats.append(env * _sph_j(l, arg) * Ps[l] * nmask8)
    sbf = jnp.concatenate(feats, axis=1)
    amb_ref[...] = sbf @ wa_ref[...]


def _angle_stage(gs, gt, wa_pad, ba):
    A = gs.shape[0]
    grid = (A // ba,)
    return pl.pallas_call(
        _angle_body,
        grid=grid,
        in_specs=[pl.BlockSpec((ba, 16), lambda i: (i, 0)),
                  pl.BlockSpec((ba, 16), lambda i: (i, 0)),
                  pl.BlockSpec((56, 32), lambda i: (0, 0))],
        out_specs=pl.BlockSpec((ba, 32), lambda i: (i, 0)),
        out_shape=jax.ShapeDtypeStruct((A, 32), jnp.float32),
        interpret=INTERPRET,
    )(gs, gt, wa_pad)


def _bilin_body(qse_ref, gt_ref, amb_ref, wd_ref, wt_ref, sm_ref, *, boff):
    rbf_te = gt_ref[...][:, 8:16]
    dmb = rbf_te @ wd_ref[...]
    smp = qse_ref[...] * dmb
    amb = amb_ref[...]
    wt = wt_ref[...]
    acc = None
    for j in range(BIL):
        term = amb[:, boff + j:boff + j + 1] * (smp @ wt[j])
        acc = term if acc is None else acc + term
    sm_ref[...] = acc


def _bilin_stage(qse, gt, amb, wd_b, wt_b, boff, ba):
    A = qse.shape[0]
    grid = (A // ba,)
    return pl.pallas_call(
        functools.partial(_bilin_body, boff=boff),
        grid=grid,
        in_specs=[pl.BlockSpec((ba, H), lambda i: (i, 0)),
                  pl.BlockSpec((ba, 16), lambda i: (i, 0)),
                  pl.BlockSpec((ba, 32), lambda i: (i, 0)),
                  pl.BlockSpec((8, H), lambda i: (0, 0)),
                  pl.BlockSpec((BIL, H, H), lambda i: (0, 0, 0))],
        out_specs=pl.BlockSpec((ba, H), lambda i: (i, 0)),
        out_shape=jax.ShapeDtypeStruct((A, H), jnp.float32),
        interpret=INTERPRET,
    )(qse, gt, amb, wd_b, wt_b)


def _emlp_body(agg_ref, msg_ref, geom_ref, wtm_ref, btm_ref,
               r1w1_ref, r1b1_ref, r1w2_ref, r1b2_ref,
               wsk_ref, bsk_ref,
               r2w1_ref, r2b1_ref, r2w2_ref, r2b2_ref,
               owd_ref, wsm_ref, bsm_ref,
               out_msg_ref, out_t_ref, out_q_ref, *, has_q):
    msg = msg_ref[...]
    x = agg_ref[...] + msg @ wtm_ref[...] + btm_ref[...]
    x = x + _silu(_silu(x @ r1w1_ref[...] + r1b1_ref[...]) @ r1w2_ref[...]
                  + r1b2_ref[...])
    x = _silu(x @ wsk_ref[...] + bsk_ref[...]) + msg
    for l in range(2):
        x = x + _silu(_silu(x @ r2w1_ref[l] + r2b1_ref[l]) @ r2w2_ref[l]
                      + r2b2_ref[l])
    out_msg_ref[...] = x
    rbf = geom_ref[...][:, 8:16]
    out_t_ref[...] = (rbf @ owd_ref[...]) * x
    if has_q:
        out_q_ref[...] = x @ wsm_ref[...] + bsm_ref[...]
    else:
        out_q_ref[...] = x


def _emlp_stage(agg, msg, geom, wtm, btm, r1w1, r1b1, r1w2, r1b2,
                wsk, bsk, r2w1, r2b1, r2w2, r2b2, owd, wsm, bsm, be):
    E = agg.shape[0]
    grid = (E // be,)
    row = lambda w: pl.BlockSpec((be, w), lambda i: (i, 0))
    wspec = lambda s: pl.BlockSpec(s, lambda i: (0,) * len(s))
    has_q = wsm is not None
    out_shapes = (
        jax.ShapeDtypeStruct((E, H), jnp.float32),
        jax.ShapeDtypeStruct((E, H), jnp.float32),
        jax.ShapeDtypeStruct((E, H), jnp.float32),
    )
    return pl.pallas_call(
        functools.partial(_emlp_body, has_q=has_q),
        grid=grid,
        in_specs=[row(H), row(H), row(16),
                  wspec((H, H)), wspec((1, H)),
                  wspec((H, H)), wspec((1, H)), wspec((H, H)), wspec((1, H)),
                  wspec((H, H)), wspec((1, H)),
                  wspec((2, H, H)), wspec((2, 1, H)), wspec((2, H, H)),
                  wspec((2, 1, H)),
                  wspec((8, H)), wspec((H, H)), wspec((1, H))],
        out_specs=(row(H), row(H), row(H)),
        out_shape=out_shapes,
        interpret=INTERPRET,
    )(agg, msg, geom, wtm, btm, r1w1, r1b1, r1w2, r1b2, wsk, bsk,
      r2w1, r2b1, r2w2, r2b2, owd,
      wsm if has_q else wtm, bsm if has_q else btm)


def _node_body(x_ref, wl_ref, bl_ref, wout_ref, out_ref, *, nb1, nparts):
    acc = None
    for b in range(nb1):
        x = x_ref[b, 0]
        for p in range(1, nparts):
            x = x + x_ref[b, p]
        for l in range(3):
            x = _silu(x @ wl_ref[b, l] + bl_ref[b, l])
        o = x @ wout_ref[b]
        acc = o if acc is None else acc + o
    out_ref[...] = acc


def _node_stage(xs, out_Wl, out_bl, wout_pad, bn):
    nb1, nparts, N, _ = xs.shape
    grid = (N // bn,)
    return pl.pallas_call(
        functools.partial(_node_body, nb1=nb1, nparts=nparts),
        grid=grid,
        in_specs=[pl.BlockSpec((nb1, nparts, bn, H), lambda i: (0, 0, i, 0)),
                  pl.BlockSpec(out_Wl.shape, lambda i: (0, 0, 0, 0)),
                  pl.BlockSpec((nb1, 3, 1, H), lambda i: (0, 0, 0, 0)),
                  pl.BlockSpec((nb1, H, 8), lambda i: (0, 0, 0))],
        out_specs=pl.BlockSpec((bn, 8), lambda i: (i, 0)),
        out_shape=jax.ShapeDtypeStruct((N, 8), jnp.float32),
        interpret=INTERPRET,
    )(xs, out_Wl, out_bl.reshape(nb1, 3, 1, H), wout_pad)


def kernel(atomic_number, position, edge_index, angle_index, emb_table,
           emb_Wd, emb_bd, emb_W, emb_b,
           int_Wd, int_Wa, int_Wsm, int_bsm, int_Wtm, int_btm, int_Wbil,
           int_res1_W1, int_res1_b1, int_res1_W2, int_res1_b2,
           int_Wskip, int_bskip,
           int_res2_W1, int_res2_b1, int_res2_W2, int_res2_b2,
           out_Wd, out_Wl, out_bl, out_Wout):
    N = position.shape[0]
    E = edge_index.shape[1]
    A = angle_index.shape[1]
    NB = int_Wd.shape[0]
    be = 1000 if E % 1000 == 0 else E
    ba = 1000 if A % 1000 == 0 else A
    bn = 1000 if N % 1000 == 0 else N

    src, dst = edge_index[0], edge_index[1]
    se, te = angle_index[0], angle_index[1]

    ptab = jnp.concatenate(
        [position, atomic_number[:, None].astype(jnp.float32),
         jnp.zeros((N, 4), jnp.float32)], axis=1)

    t0, t1, wrd, brd, owd0 = _prep(emb_table, emb_W, emb_Wd, emb_bd, emb_b,
                                   out_Wd[0])

    if INTERPRET:
        ps = jnp.take(ptab, src, axis=0)
        pd = jnp.take(ptab, dst, axis=0)
    else:
        ps, pd = _make_sc_gather(N, 8, E, 2)(ptab, ptab, src, dst)

    wsm0 = int_Wsm[0]
    bsm0 = int_bsm[0].reshape(1, H)
    msg, geom, t_cur, q = _edge_stage(ps, pd, t0, t1, wrd, brd, owd0,
                                      wsm0, bsm0, be)

    if INTERPRET:
        gs = jnp.take(geom, se, axis=0)
        gt = jnp.take(geom, te, axis=0)
    else:
        gs, gt = _make_sc_gather(E, 16, A, 2)(geom, geom, se, te)

    wa = int_Wa.reshape(NB, S, R, BIL)
    wa = jnp.pad(wa, ((0, 0), (0, 0), (0, 2), (0, 0)))
    wa = wa.reshape(NB, 56, BIL).transpose(1, 0, 2).reshape(56, NB * BIL)
    wa_pad = jnp.pad(wa, ((0, 0), (0, 32 - NB * BIL)))
    amb = _angle_stage(gs, gt, wa_pad, ba)

    wt = int_Wbil.transpose(0, 2, 3, 1)
    wd8 = jnp.pad(int_Wd, ((0, 0), (0, 8 - R), (0, 0)))
    owd8 = jnp.pad(out_Wd, ((0, 0), (0, 8 - R), (0, 0)))
    wout_pad = jnp.pad(out_Wout, ((0, 0), (0, 0), (0, 8 - out_Wout.shape[2])))

    qse_gather = None if INTERPRET else _make_sc_gather(E, H, A, 1)
    t_list = [t_cur]
    for b in range(NB):
        if INTERPRET:
            qse = jnp.take(q, se, axis=0)
        else:
            (qse,) = qse_gather(q, se)
        sm = _bilin_stage(qse, gt, amb, wd8[b], wt[b], BIL * b, ba)
        agg = jax.ops.segment_sum(sm, te, num_segments=E)
        has_q = b + 1 < NB
        msg, t_cur, q = _emlp_stage(
            agg, msg, geom, int_Wtm[b], int_btm[b].reshape(1, H),
            int_res1_W1[b, 0], int_res1_b1[b, 0].reshape(1, H),
            int_res1_W2[b, 0], int_res1_b2[b, 0].reshape(1, H),
            int_Wskip[b], int_bskip[b].reshape(1, H),
            int_res2_W1[b], int_res2_b1[b].reshape(2, 1, H),
            int_res2_W2[b], int_res2_b2[b].reshape(2, 1, H),
            owd8[b + 1],
            int_Wsm[b + 1] if has_q else None,
            int_bsm[b + 1].reshape(1, H) if has_q else None, be)
        t_list.append(t_cur)

    if INTERPRET:
        xs = jnp.stack([jax.ops.segment_sum(t, dst, num_segments=N)[None]
                        for t in t_list], axis=0)
    else:
        scatter_n = _make_sc_scatter_n(N, H, E)
        zn = jnp.zeros((N // _SC_NS, H), jnp.float32)
        xs = jnp.stack([scatter_n(t, dst, zn) for t in t_list], axis=0)
    out = _node_stage(xs, out_Wl, out_bl, wout_pad, bn)
    return out[:, 0:1]
